# baseline (speedup 1.0000x reference)
import functools

import jax
import jax.numpy as jnp
from jax import lax
from jax.experimental import pallas as pl
from jax.experimental.pallas import tpu as pltpu
from jax.experimental.pallas import tpu_sc as plsc

B = 16384
D = 16
NC = 2
NS = 16
L = 16
NW = NC * NS
BPW = B // NW
CH = 128
NCH = BPW // CH
GROUPS = BPW // L

_mesh = plsc.VectorSubcoreMesh(
    core_axis_name="c", subcore_axis_name="s", num_cores=NC, num_subcores=NS
)


@functools.partial(
    pl.kernel,
    out_type=jax.ShapeDtypeStruct((B,), jnp.float32),
    mesh=_mesh,
    scratch_types=[
        pltpu.VMEM((NCH, CH), jnp.int32),
        pltpu.VMEM((NCH, CH), jnp.int32),
        pltpu.VMEM((BPW, D), jnp.float32),
        pltpu.VMEM((BPW, D), jnp.float32),
        pltpu.VMEM((BPW,), jnp.float32),
        pltpu.SemaphoreType.DMA,
        pltpu.SemaphoreType.DMA,
    ],
    compiler_params=pltpu.CompilerParams(
        needs_layout_passes=False, use_tc_tiling_on_sc=False),
)
def _mf(v_hbm, j_hbm, hu_hbm, hi_hbm, out_hbm,
        vidx, jidx, hu_v, hi_v, out_v, sem_u, sem_i):
    wid = lax.axis_index("s") * NC + lax.axis_index("c")
    base = wid * BPW

    pltpu.sync_copy(v_hbm.at[pl.ds(wid * NCH, NCH)], vidx)
    pltpu.sync_copy(j_hbm.at[pl.ds(wid * NCH, NCH)], jidx)

    copies = []
    for k in range(NCH):
        copies.append(pltpu.async_copy(
            hu_hbm.at[vidx.at[k]], hu_v.at[pl.ds(k * CH, CH)], sem_u))
        copies.append(pltpu.async_copy(
            hi_hbm.at[jidx.at[k]], hi_v.at[pl.ds(k * CH, CH)], sem_i))
    for c in copies:
        c.wait()

    lane = lax.iota(jnp.int32, L)

    def group_body(g, carry):
        rows = g * L + lane
        acc = jnp.zeros((L,), jnp.float32)
        for d in range(D):
            col = jnp.full((L,), d, jnp.int32)
            hu = plsc.load_gather(hu_v, [rows, col])
            hi = plsc.load_gather(hi_v, [rows, col])
            acc = acc + hu * hi
        out_v[pl.ds(g * L, L)] = acc
        return carry

    lax.fori_loop(0, GROUPS, group_body, 0)

    pltpu.sync_copy(out_v, out_hbm.at[pl.ds(base, BPW)])


def kernel(u, i, r, v, j, h_u, h_i):
    del u, i, r
    v2d = v.astype(jnp.int32).reshape(NW * NCH, CH)
    j2d = j.astype(jnp.int32).reshape(NW * NCH, CH)
    return _mf(v2d, j2d, h_u, h_i)
